# baseline (speedup 1.0000x reference)
import functools

import jax
import jax.numpy as jnp
from jax import lax
from jax.experimental import pallas as pl
from jax.experimental.pallas import tpu as pltpu
from jax.experimental.pallas import tpu_sc as plsc

D_MODEL = 768
SEQ_LEN = 8192
BATCH = 4

_info = plsc.get_sparse_core_info()
_NC = _info.num_cores
_NS = _info.num_subcores
_NW = _NC * _NS
_B_TOTAL = BATCH * SEQ_LEN
_B_PER_W = _B_TOTAL // _NW
_CHUNK = 32
_N_CHUNKS = _B_PER_W // _CHUNK
_NBUF = 4

_mesh = plsc.VectorSubcoreMesh(core_axis_name="c", subcore_axis_name="s")


@functools.partial(
    pl.kernel,
    mesh=_mesh,
    out_type=jax.ShapeDtypeStruct((_B_TOTAL, D_MODEL), jnp.float32),
    scratch_types=(
        [pltpu.VMEM((_N_CHUNKS, _CHUNK), jnp.int32),
         pltpu.VMEM((_NBUF, _CHUNK, D_MODEL), jnp.float32)]
        + [pltpu.SemaphoreType.DMA] * (2 * _NBUF)
    ),
)
def _gather_kernel(idx_hbm, table_hbm, out_hbm, idx_v, rows_v, *sems):
    gsems = sems[:_NBUF]
    ssems = sems[_NBUF:]
    wid = lax.axis_index("s") * _NC + lax.axis_index("c")
    base = wid * _B_PER_W
    pltpu.sync_copy(idx_hbm.at[wid], idx_v)
    gathers = [None] * _NBUF
    scatters = [None] * _NBUF
    for j in range(_NBUF - 1):
        gathers[j] = pltpu.async_copy(
            table_hbm.at[idx_v.at[j]], rows_v.at[j], gsems[j])
    for j in range(_N_CHUNKS):
        cb = j % _NBUF
        nb = (j + _NBUF - 1) % _NBUF
        jn = j + _NBUF - 1
        if jn < _N_CHUNKS:
            if scatters[nb] is not None:
                scatters[nb].wait()
            gathers[nb] = pltpu.async_copy(
                table_hbm.at[idx_v.at[jn]], rows_v.at[nb], gsems[nb])
        gathers[cb].wait()
        scatters[cb] = pltpu.async_copy(
            rows_v.at[cb], out_hbm.at[pl.ds(base + j * _CHUNK, _CHUNK)],
            ssems[cb])
    for b in range(_NBUF):
        if scatters[b] is not None:
            scatters[b].wait()


def kernel(x, pe):
    idx = x.reshape(_NW, _N_CHUNKS, _CHUNK)
    out = _gather_kernel(idx, pe)
    return out.reshape(BATCH, SEQ_LEN, D_MODEL)
